# baseline (speedup 1.0000x reference)
import jax
import jax.numpy as jnp
from jax import lax
from jax.experimental import pallas as pl
from jax.experimental.pallas import tpu as pltpu
from jax.experimental.pallas import tpu_sc as plsc

_N = 240
_NC, _NS = 2, 16
_NW = _NC * _NS
_ROWS = _N * _N
_RPW = _ROWS // _NW
_L = 16


def _mbuild(eflat_hbm, out_hbm, eflat_v, acc_v):
    cid = lax.axis_index("c")
    sid = lax.axis_index("s")
    wid = cid * _NS + sid
    lo = wid * _RPW
    ep = eflat_v.shape[0]
    pltpu.sync_copy(eflat_hbm, eflat_v)

    nacc = acc_v.shape[0]

    def _zero(i, carry):
        acc_v[pl.ds(i * _L, _L)] = jnp.zeros((_L,), jnp.float32)
        return carry

    lax.fori_loop(0, nacc // _L, _zero, 0, unroll=8)

    ones = jnp.full((_L,), 1.0, jnp.float32)

    def _scatter(i, carry):
        v = eflat_v[pl.ds(i * _L, _L)]
        local = v - lo
        mask = (local >= 0) & (local < _RPW)
        idx = jnp.where(mask, local, _RPW)
        plsc.addupdate_scatter(acc_v, [idx], ones)
        return carry

    lax.fori_loop(0, ep // _L, _scatter, 0, unroll=4)
    pltpu.sync_copy(acc_v.at[pl.ds(0, _RPW)], out_hbm.at[pl.ds(lo, _RPW)])


def _build_m_flat(edge_index):
    ne = edge_index.shape[1]
    flat = edge_index[1] * _N + edge_index[0]
    ep = ((ne + _L - 1) // _L) * _L
    flat = jnp.pad(flat, ((0, ep - ne)), constant_values=_ROWS)
    mesh = plsc.VectorSubcoreMesh(core_axis_name="c", subcore_axis_name="s")
    return pl.kernel(
        _mbuild,
        out_type=jax.ShapeDtypeStruct((_ROWS,), jnp.float32),
        mesh=mesh,
        compiler_params=pltpu.CompilerParams(needs_layout_passes=False),
        scratch_types=[
            pltpu.VMEM((ep,), jnp.int32),
            pltpu.VMEM((_RPW + 2 * _L - _RPW % _L,), jnp.float32),
        ],
    )(flat)


def _step(mp_ref, inp_ref, st_ref, wgs_ref, wgi_ref, asrc_ref,
          adst_ref, bg_ref, w1ir_ref, w1hr_ref, w1iu_ref, w1hu_ref,
          b1r_ref, b1u_ref, w2i_ref, w2h_ref, b2_ref, out_ref, m_scr):
    b = pl.program_id(0)
    f32 = jnp.float32

    @pl.when(b == 0)
    def _build_m():
        row = lax.broadcasted_iota(jnp.int32, (_N, _N), 0)
        col = lax.broadcasted_iota(jnp.int32, (_N, _N), 1)
        m_scr[...] = mp_ref[...] + (row == col).astype(f32)

    inp = inp_ref[0]
    st = st_ref[0]
    h = (jnp.dot(st, wgs_ref[...], preferred_element_type=f32)
         + jnp.dot(inp, wgi_ref[...], preferred_element_type=f32))
    a_d = jnp.dot(h, adst_ref[...], preferred_element_type=f32)
    a_s = lax.dot_general(asrc_ref[...], h, (((1,), (1,)), ((), ())),
                          preferred_element_type=f32)
    e = a_d + a_s
    e = jnp.where(e >= 0, e, 0.2 * e)
    m = m_scr[...]
    mask = m > 0
    em = jnp.where(mask, e, -1e30)
    rowmax = jnp.max(em, axis=1, keepdims=True)
    p = m * jnp.exp(em - rowmax)
    denom = jnp.sum(p, axis=1, keepdims=True)
    alpha = p / (denom + 1e-16)
    s2 = jnp.dot(alpha, h, preferred_element_type=f32) + bg_ref[...]
    r = jax.nn.sigmoid(jnp.dot(inp, w1ir_ref[...], preferred_element_type=f32)
                       + jnp.dot(s2, w1hr_ref[...], preferred_element_type=f32)
                       + b1r_ref[...])
    u = jax.nn.sigmoid(jnp.dot(inp, w1iu_ref[...], preferred_element_type=f32)
                       + jnp.dot(s2, w1hu_ref[...], preferred_element_type=f32)
                       + b1u_ref[...])
    c = jnp.tanh(jnp.dot(inp, w2i_ref[...], preferred_element_type=f32)
                 + jnp.dot(r * s2, w2h_ref[...], preferred_element_type=f32)
                 + b2_ref[...])
    out_ref[0] = u * s2 + (1.0 - u) * c


def kernel(inputs, state, edge_index, W_gat, att_src, att_dst, b_gat, bias1,
           W1, b1, W2, b2):
    B = inputs.shape[0]
    U = att_src.shape[0]
    F = W_gat.shape[0] - U
    N = _N

    mp = _build_m_flat(edge_index).reshape(N, N)

    inp3 = inputs.reshape(B, N, F)
    st3 = state.reshape(B, N, U)

    wgs = W_gat[:U]
    wgi = W_gat[U:]
    asrc = att_src.reshape(1, U)
    adst = att_dst.reshape(U, 1)
    bg = (b_gat + bias1).reshape(1, U)
    w1ir, w1iu = W1[:F, :U], W1[:F, U:]
    w1hr, w1hu = W1[F:, :U], W1[F:, U:]
    b1r, b1u = b1[:U].reshape(1, U), b1[U:].reshape(1, U)
    w2i, w2h = W2[:F], W2[F:]
    b2r = b2.reshape(1, U)

    def cmap(*shape):
        return pl.BlockSpec(shape, lambda b: (0,) * len(shape))

    out = pl.pallas_call(
        _step,
        grid=(B,),
        in_specs=[
            cmap(N, N),
            pl.BlockSpec((1, N, F), lambda b: (b, 0, 0)),
            pl.BlockSpec((1, N, U), lambda b: (b, 0, 0)),
            cmap(U, U), cmap(F, U), cmap(1, U), cmap(U, 1), cmap(1, U),
            cmap(F, U), cmap(U, U), cmap(F, U), cmap(U, U),
            cmap(1, U), cmap(1, U), cmap(F, U), cmap(U, U), cmap(1, U),
        ],
        out_specs=pl.BlockSpec((1, N, U), lambda b: (b, 0, 0)),
        out_shape=jax.ShapeDtypeStruct((B, N, U), jnp.float32),
        scratch_shapes=[pltpu.VMEM((N, N), jnp.float32)],
    )(mp, inp3, st3, wgs, wgi, asrc, adst, bg,
      w1ir, w1hr, w1iu, w1hu, b1r, b1u, w2i, w2h, b2r)
    return out.reshape(B, N * U)


# speedup vs baseline: 1.3341x; 1.3341x over previous
import jax
import jax.numpy as jnp
from jax import lax
from jax.experimental import pallas as pl
from jax.experimental.pallas import tpu as pltpu
from jax.experimental.pallas import tpu_sc as plsc

_N = 240
_NC, _NS = 2, 16
_NG = _N // 8
_RPW = 2 * 8 * 128
_L = 16


def _mbuild(eflat_hbm, out_hbm, eflat_v, acc_v):
    cid = lax.axis_index("c")
    sid = lax.axis_index("s")
    wid = cid * _NS + sid
    lo = wid * _RPW
    ep = eflat_v.shape[0]
    pltpu.sync_copy(eflat_hbm, eflat_v)

    nacc = acc_v.shape[0]

    def _zero(i, carry):
        acc_v[pl.ds(i * _L, _L)] = jnp.zeros((_L,), jnp.float32)
        return carry

    lax.fori_loop(0, nacc // _L, _zero, 0, unroll=8)

    ones = jnp.full((_L,), 1.0, jnp.float32)

    def _scatter(i, carry):
        v = eflat_v[pl.ds(i * _L, _L)]
        local = v - lo
        mask = (local >= 0) & (local < _RPW)
        idx = jnp.where(mask, local, _RPW)
        plsc.addupdate_scatter(acc_v, [idx], ones)
        return carry

    lax.fori_loop(0, ep // _L, _scatter, 0, unroll=4)

    @pl.when(wid < _NG)
    def _out():
        pltpu.sync_copy(acc_v.at[pl.ds(0, _RPW)], out_hbm.at[pl.ds(lo, _RPW)])


def _build_m_tiled(edge_index):
    ne = edge_index.shape[1]
    s, d = edge_index[0], edge_index[1]
    tflat = (d >> 3) * _RPW + (s >> 7) * 1024 + (d & 7) * 128 + (s & 127)
    ep = ((ne + _L - 1) // _L) * _L
    tflat = jnp.pad(tflat, ((0, ep - ne)), constant_values=_NG * _RPW)
    mesh = plsc.VectorSubcoreMesh(core_axis_name="c", subcore_axis_name="s")
    out = pl.kernel(
        _mbuild,
        out_type=jax.ShapeDtypeStruct((_NG * _RPW,), jnp.float32),
        mesh=mesh,
        compiler_params=pltpu.CompilerParams(needs_layout_passes=False),
        scratch_types=[
            pltpu.VMEM((ep,), jnp.int32),
            pltpu.VMEM((_RPW + _L,), jnp.float32),
        ],
    )(tflat)
    return out.reshape(_NG, 2, 8, 128)


_BP = 4


def _step(mp_ref, inp_ref, st_ref, wgs_ref, wgi_ref, asrc_ref,
          adst_ref, bg_ref, w1ir_ref, w1hr_ref, w1iu_ref, w1hu_ref,
          b1r_ref, b1u_ref, w2i_ref, w2h_ref, b2_ref, out_ref, m_scr):
    b = pl.program_id(0)
    f32 = jnp.float32

    @pl.when(b == 0)
    def _build_m():
        left = mp_ref[:, 0].reshape(_N, 128)
        right = mp_ref[:, 1].reshape(_N, 128)
        row = lax.broadcasted_iota(jnp.int32, (_N, 128), 0)
        col = lax.broadcasted_iota(jnp.int32, (_N, 128), 1)
        m_scr[:, 0:128] = left + (row == col).astype(f32)
        m_scr[:, 128:_N] = (right + (row == col + 128).astype(f32))[:, 0:_N - 128]

    nf = inp_ref.shape[2]
    nu = st_ref.shape[2]
    inp = inp_ref[...].reshape(_BP * _N, nf).astype(f32)
    st = st_ref[...].reshape(_BP * _N, nu).astype(f32)
    h = (jnp.dot(st, wgs_ref[...], preferred_element_type=f32)
         + jnp.dot(inp, wgi_ref[...], preferred_element_type=f32))
    a_d = jnp.dot(h, adst_ref[...], preferred_element_type=f32)
    m = m_scr[...]
    mask = m > 0
    s2parts = []
    for j in range(_BP):
        hj = h[j * _N:(j + 1) * _N]
        a_s = lax.dot_general(asrc_ref[...], hj, (((1,), (1,)), ((), ())),
                              preferred_element_type=f32)
        e = a_d[j * _N:(j + 1) * _N] + a_s
        e = jnp.where(e >= 0, e, 0.2 * e)
        em = jnp.where(mask, e, -1e30)
        rowmax = jnp.max(em, axis=1, keepdims=True)
        p = m * jnp.exp(em - rowmax)
        denom = jnp.sum(p, axis=1, keepdims=True)
        alpha = p / (denom + 1e-16)
        s2parts.append(jnp.dot(alpha, hj, preferred_element_type=f32))
    s2 = jnp.concatenate(s2parts, axis=0) + bg_ref[...]
    r = jax.nn.sigmoid(jnp.dot(inp, w1ir_ref[...], preferred_element_type=f32)
                       + jnp.dot(s2, w1hr_ref[...], preferred_element_type=f32)
                       + b1r_ref[...])
    u = jax.nn.sigmoid(jnp.dot(inp, w1iu_ref[...], preferred_element_type=f32)
                       + jnp.dot(s2, w1hu_ref[...], preferred_element_type=f32)
                       + b1u_ref[...])
    c = jnp.tanh(jnp.dot(inp, w2i_ref[...], preferred_element_type=f32)
                 + jnp.dot(r * s2, w2h_ref[...], preferred_element_type=f32)
                 + b2_ref[...])
    out_ref[...] = (u * s2 + (1.0 - u) * c).reshape(_BP, _N, nu)


def kernel(inputs, state, edge_index, W_gat, att_src, att_dst, b_gat, bias1,
           W1, b1, W2, b2):
    B = inputs.shape[0]
    U = att_src.shape[0]
    F = W_gat.shape[0] - U
    N = _N

    mp = _build_m_tiled(edge_index)

    inp3 = inputs.astype(jnp.bfloat16).reshape(B, N, F)
    st3 = state.astype(jnp.bfloat16).reshape(B, N, U)

    wgs = W_gat[:U]
    wgi = W_gat[U:]
    asrc = att_src.reshape(1, U)
    adst = att_dst.reshape(U, 1)
    bg = (b_gat + bias1).reshape(1, U)
    w1ir, w1iu = W1[:F, :U], W1[:F, U:]
    w1hr, w1hu = W1[F:, :U], W1[F:, U:]
    b1r, b1u = b1[:U].reshape(1, U), b1[U:].reshape(1, U)
    w2i, w2h = W2[:F], W2[F:]
    b2r = b2.reshape(1, U)

    def cmap(*shape):
        return pl.BlockSpec(shape, lambda b: (0,) * len(shape))

    out = pl.pallas_call(
        _step,
        grid=(B // _BP,),
        in_specs=[
            cmap(_NG, 2, 8, 128),
            pl.BlockSpec((_BP, N, F), lambda b: (b, 0, 0)),
            pl.BlockSpec((_BP, N, U), lambda b: (b, 0, 0)),
            cmap(U, U), cmap(F, U), cmap(1, U), cmap(U, 1), cmap(1, U),
            cmap(F, U), cmap(U, U), cmap(F, U), cmap(U, U),
            cmap(1, U), cmap(1, U), cmap(F, U), cmap(U, U), cmap(1, U),
        ],
        out_specs=pl.BlockSpec((_BP, N, U), lambda b: (b, 0, 0)),
        out_shape=jax.ShapeDtypeStruct((B, N, U), jnp.float32),
        scratch_shapes=[pltpu.VMEM((N, N), jnp.float32)],
    )(mp, inp3, st3, wgs, wgi, asrc, adst, bg,
      w1ir, w1hr, w1iu, w1hu, b1r, b1u, w2i, w2h, b2r)
    return out.reshape(B, N * U)


# speedup vs baseline: 1.6229x; 1.2164x over previous
import jax
import jax.numpy as jnp
from jax import lax
from jax.experimental import pallas as pl
from jax.experimental.pallas import tpu as pltpu
from jax.experimental.pallas import tpu_sc as plsc

_N = 240
_NC, _NS = 2, 16
_NG = _N // 8
_RPW = 2 * 8 * 128
_L = 16


def _mbuild(eflat_hbm, out_hbm, eflat_v, acc_v):
    cid = lax.axis_index("c")
    sid = lax.axis_index("s")
    wid = cid * _NS + sid
    lo = wid * _RPW
    ep = eflat_v.shape[0]
    pltpu.sync_copy(eflat_hbm, eflat_v)

    nacc = acc_v.shape[0]

    def _zero(i, carry):
        acc_v[pl.ds(i * _L, _L)] = jnp.zeros((_L,), jnp.float32)
        return carry

    lax.fori_loop(0, nacc // _L, _zero, 0, unroll=8)

    ones = jnp.full((_L,), 1.0, jnp.float32)

    def _scatter(i, carry):
        v = eflat_v[pl.ds(i * _L, _L)]
        local = v - lo
        mask = (local >= 0) & (local < _RPW)
        idx = jnp.where(mask, local, _RPW)
        plsc.addupdate_scatter(acc_v, [idx], ones)
        return carry

    lax.fori_loop(0, ep // _L, _scatter, 0, unroll=4)

    @pl.when(wid < _NG)
    def _out():
        pltpu.sync_copy(acc_v.at[pl.ds(0, _RPW)], out_hbm.at[pl.ds(lo, _RPW)])


def _build_m_tiled(edge_index):
    ne = edge_index.shape[1]
    s, d = edge_index[0], edge_index[1]
    tflat = (d >> 3) * _RPW + (s >> 7) * 1024 + (d & 7) * 128 + (s & 127)
    ep = ((ne + _L - 1) // _L) * _L
    tflat = jnp.pad(tflat, ((0, ep - ne)), constant_values=_NG * _RPW)
    mesh = plsc.VectorSubcoreMesh(core_axis_name="c", subcore_axis_name="s")
    out = pl.kernel(
        _mbuild,
        out_type=jax.ShapeDtypeStruct((_NG * _RPW,), jnp.float32),
        mesh=mesh,
        compiler_params=pltpu.CompilerParams(needs_layout_passes=False),
        scratch_types=[
            pltpu.VMEM((ep,), jnp.int32),
            pltpu.VMEM((_RPW + _L,), jnp.float32),
        ],
    )(tflat)
    return out.reshape(_NG, 2, 8, 128)


_BP = 4


def _step(mp_ref, inp_ref, st_ref, wgs_ref, wgi_ref, asrc_ref,
          adst_ref, bg_ref, w1ir_ref, w1hr_ref, w1iu_ref, w1hu_ref,
          b1r_ref, b1u_ref, w2i_ref, w2h_ref, b2_ref, out_ref, m_scr):
    b = pl.program_id(0)
    f32 = jnp.float32

    @pl.when(b == 0)
    def _build_m():
        epp = mp_ref.shape[0]
        iota_n = lax.broadcasted_iota(jnp.int32, (epp, _N), 1)
        oh_s = (mp_ref[:, 0:1] == iota_n).astype(f32)
        oh_d = (mp_ref[:, 1:2] == iota_n).astype(f32)
        mm = lax.dot_general(oh_d, oh_s, (((0,), (0,)), ((), ())),
                             preferred_element_type=f32)
        row = lax.broadcasted_iota(jnp.int32, (_N, _N), 0)
        col = lax.broadcasted_iota(jnp.int32, (_N, _N), 1)
        m_scr[...] = mm + (row == col).astype(f32)

    nf = inp_ref.shape[2]
    nu = st_ref.shape[2]
    inp = inp_ref[...].reshape(_BP * _N, nf).astype(f32)
    st = st_ref[...].reshape(_BP * _N, nu).astype(f32)
    h = (jnp.dot(st, wgs_ref[...], preferred_element_type=f32)
         + jnp.dot(inp, wgi_ref[...], preferred_element_type=f32))
    a_d = jnp.dot(h, adst_ref[...], preferred_element_type=f32)
    m = m_scr[...]
    mask = m > 0
    s2parts = []
    for j in range(_BP):
        hj = h[j * _N:(j + 1) * _N]
        a_s = lax.dot_general(asrc_ref[...], hj, (((1,), (1,)), ((), ())),
                              preferred_element_type=f32)
        e = a_d[j * _N:(j + 1) * _N] + a_s
        e = jnp.where(e >= 0, e, 0.2 * e)
        em = jnp.where(mask, e, -1e30)
        rowmax = jnp.max(em, axis=1, keepdims=True)
        p = m * jnp.exp(em - rowmax)
        denom = jnp.sum(p, axis=1, keepdims=True)
        alpha = p / (denom + 1e-16)
        s2parts.append(jnp.dot(alpha, hj, preferred_element_type=f32))
    s2 = jnp.concatenate(s2parts, axis=0) + bg_ref[...]
    r = jax.nn.sigmoid(jnp.dot(inp, w1ir_ref[...], preferred_element_type=f32)
                       + jnp.dot(s2, w1hr_ref[...], preferred_element_type=f32)
                       + b1r_ref[...])
    u = jax.nn.sigmoid(jnp.dot(inp, w1iu_ref[...], preferred_element_type=f32)
                       + jnp.dot(s2, w1hu_ref[...], preferred_element_type=f32)
                       + b1u_ref[...])
    c = jnp.tanh(jnp.dot(inp, w2i_ref[...], preferred_element_type=f32)
                 + jnp.dot(r * s2, w2h_ref[...], preferred_element_type=f32)
                 + b2_ref[...])
    out_ref[...] = (u * s2 + (1.0 - u) * c).reshape(_BP, _N, nu)


def kernel(inputs, state, edge_index, W_gat, att_src, att_dst, b_gat, bias1,
           W1, b1, W2, b2):
    B = inputs.shape[0]
    U = att_src.shape[0]
    F = W_gat.shape[0] - U
    N = _N

    ne = edge_index.shape[1]
    epq = ((ne + 7) // 8) * 8
    mp = jnp.pad(edge_index.T, ((0, epq - ne), (0, 0)), constant_values=-1)

    inp3 = inputs.astype(jnp.bfloat16).reshape(B, N, F)
    st3 = state.astype(jnp.bfloat16).reshape(B, N, U)

    wgs = W_gat[:U]
    wgi = W_gat[U:]
    asrc = att_src.reshape(1, U)
    adst = att_dst.reshape(U, 1)
    bg = (b_gat + bias1).reshape(1, U)
    w1ir, w1iu = W1[:F, :U], W1[:F, U:]
    w1hr, w1hu = W1[F:, :U], W1[F:, U:]
    b1r, b1u = b1[:U].reshape(1, U), b1[U:].reshape(1, U)
    w2i, w2h = W2[:F], W2[F:]
    b2r = b2.reshape(1, U)

    def cmap(*shape):
        return pl.BlockSpec(shape, lambda b: (0,) * len(shape))

    out = pl.pallas_call(
        _step,
        grid=(B // _BP,),
        in_specs=[
            cmap(((edge_index.shape[1] + 7) // 8) * 8, 2),
            pl.BlockSpec((_BP, N, F), lambda b: (b, 0, 0)),
            pl.BlockSpec((_BP, N, U), lambda b: (b, 0, 0)),
            cmap(U, U), cmap(F, U), cmap(1, U), cmap(U, 1), cmap(1, U),
            cmap(F, U), cmap(U, U), cmap(F, U), cmap(U, U),
            cmap(1, U), cmap(1, U), cmap(F, U), cmap(U, U), cmap(1, U),
        ],
        out_specs=pl.BlockSpec((_BP, N, U), lambda b: (b, 0, 0)),
        out_shape=jax.ShapeDtypeStruct((B, N, U), jnp.float32),
        scratch_shapes=[pltpu.VMEM((N, N), jnp.float32)],
    )(mp, inp3, st3, wgs, wgi, asrc, adst, bg,
      w1ir, w1hr, w1iu, w1hu, b1r, b1u, w2i, w2h, b2r)
    return out.reshape(B, N * U)
